# baseline (speedup 1.0000x reference)
import jax
import jax.numpy as jnp
from jax.experimental import pallas as pl

_EPS = 1e-9
_R = 8


def _diversity_sum_kernel(xt_ref, yt_ref, xf_ref, yf_ref, out_ref):
    b = pl.program_id(0)
    xt = xt_ref[0]
    yt = yt_ref[0]
    xf = xf_ref[0]
    yf = yf_ref[0]
    C, W = xt.shape
    nc = C // _R
    acc1 = jnp.zeros((_R, W), jnp.float32)
    acc2 = jnp.zeros((_R, W), jnp.float32)
    for cj in range(nc):
        xj = jnp.broadcast_to(xf[cj:cj + 1, :], (_R, W))
        yj = jnp.broadcast_to(yf[cj:cj + 1, :], (_R, W))
        for ci in range(cj + 1):
            dx = xt[ci * _R:(ci + 1) * _R, :] - xj
            dy = yt[ci * _R:(ci + 1) * _R, :] - yj
            s2 = dx * dx + dy * dy + _EPS
            d = s2 * jax.lax.rsqrt(s2)
            if ci == cj:
                acc1 = acc1 + d
            else:
                acc2 = acc2 + d
    s = 2.0 * jnp.sum(acc2) + jnp.sum(acc1)

    @pl.when(b == 0)
    def _():
        out_ref[:, :] = jnp.zeros_like(out_ref)

    out_ref[:, :] = out_ref[:, :] + s


def kernel(predicted_trajectory):
    traj = predicted_trajectory[..., :2]
    B, C, T = traj.shape[:3]
    x = traj[..., 0]
    y = traj[..., 1]
    W = _R * T
    nc = C // _R
    xt = jnp.tile(x, (1, 1, _R))
    yt = jnp.tile(y, (1, 1, _R))
    xf = x.reshape(B, nc, W)
    yf = y.reshape(B, nc, W)
    total = pl.pallas_call(
        _diversity_sum_kernel,
        grid=(B,),
        in_specs=[
            pl.BlockSpec((1, C, W), lambda b: (b, 0, 0)),
            pl.BlockSpec((1, C, W), lambda b: (b, 0, 0)),
            pl.BlockSpec((1, nc, W), lambda b: (b, 0, 0)),
            pl.BlockSpec((1, nc, W), lambda b: (b, 0, 0)),
        ],
        out_specs=pl.BlockSpec((1, 1), lambda b: (0, 0)),
        out_shape=jax.ShapeDtypeStruct((1, 1), jnp.float32),
    )(xt, yt, xf, yf)[0, 0]
    offdiag = total / T - B * C * jnp.sqrt(jnp.float32(_EPS))
    return -(offdiag / (B * C * (C - 1)))


# speedup vs baseline: 2.0708x; 2.0708x over previous
import jax
import jax.numpy as jnp
from jax.experimental import pallas as pl

_EPS = 1e-9
_R = 8


def _diversity_sum_kernel(x_ref, y_ref, xf_ref, yf_ref, out_ref):
    b = pl.program_id(0)
    x = x_ref[0]
    y = y_ref[0]
    xf = xf_ref[0]
    yf = yf_ref[0]
    xt = jnp.concatenate([x] * _R, axis=1)
    yt = jnp.concatenate([y] * _R, axis=1)
    C, W = xt.shape
    nc = C // _R
    acc1 = jnp.zeros((_R, W), jnp.float32)
    acc2 = jnp.zeros((_R, W), jnp.float32)
    for cj in range(nc):
        xj = jnp.broadcast_to(xf[cj:cj + 1, :], (_R, W))
        yj = jnp.broadcast_to(yf[cj:cj + 1, :], (_R, W))
        for ci in range(cj + 1):
            dx = xt[ci * _R:(ci + 1) * _R, :] - xj
            dy = yt[ci * _R:(ci + 1) * _R, :] - yj
            s2 = dx * dx + dy * dy + _EPS
            d = s2 * jax.lax.rsqrt(s2)
            if ci == cj:
                acc1 = acc1 + d
            else:
                acc2 = acc2 + d
    s = 2.0 * jnp.sum(acc2) + jnp.sum(acc1)

    @pl.when(b == 0)
    def _():
        out_ref[:, :] = jnp.zeros_like(out_ref)

    out_ref[:, :] = out_ref[:, :] + s


def kernel(predicted_trajectory):
    traj = predicted_trajectory[..., :2]
    B, C, T = traj.shape[:3]
    x = traj[..., 0]
    y = traj[..., 1]
    W = _R * T
    nc = C // _R
    xf = x.reshape(B, nc, W)
    yf = y.reshape(B, nc, W)
    total = pl.pallas_call(
        _diversity_sum_kernel,
        grid=(B,),
        in_specs=[
            pl.BlockSpec((1, C, T), lambda b: (b, 0, 0)),
            pl.BlockSpec((1, C, T), lambda b: (b, 0, 0)),
            pl.BlockSpec((1, nc, W), lambda b: (b, 0, 0)),
            pl.BlockSpec((1, nc, W), lambda b: (b, 0, 0)),
        ],
        out_specs=pl.BlockSpec((1, 1), lambda b: (0, 0)),
        out_shape=jax.ShapeDtypeStruct((1, 1), jnp.float32),
    )(x, y, xf, yf)[0, 0]
    offdiag = total / T - B * C * jnp.sqrt(jnp.float32(_EPS))
    return -(offdiag / (B * C * (C - 1)))


# speedup vs baseline: 3.5991x; 1.7380x over previous
import jax
import jax.numpy as jnp
from jax.experimental import pallas as pl

_EPS = 1e-9
_R = 8
_G = 8


def _diversity_sum_kernel(x_ref, y_ref, xf_ref, yf_ref, out_ref):
    b = pl.program_id(0)
    x = x_ref[...]
    y = y_ref[...]
    xf = xf_ref[...]
    yf = yf_ref[...]
    xt = jnp.concatenate([x] * _R, axis=2)
    yt = jnp.concatenate([y] * _R, axis=2)
    G, C, W = xt.shape
    nc = C // _R
    acc1 = jnp.zeros((G, _R, W), jnp.float32)
    acc2 = jnp.zeros((G, _R, W), jnp.float32)
    for cj in range(nc):
        xj = xf[:, cj:cj + 1, :]
        yj = yf[:, cj:cj + 1, :]
        for ci in range(cj + 1):
            dx = xt[:, ci * _R:(ci + 1) * _R, :] - xj
            dy = yt[:, ci * _R:(ci + 1) * _R, :] - yj
            s2 = dx * dx + dy * dy + _EPS
            d = s2 * jax.lax.rsqrt(s2)
            if ci == cj:
                acc1 = acc1 + d
            else:
                acc2 = acc2 + d
    s = 2.0 * jnp.sum(acc2) + jnp.sum(acc1)

    @pl.when(b == 0)
    def _():
        out_ref[:, :] = jnp.zeros_like(out_ref)

    out_ref[:, :] = out_ref[:, :] + s


def kernel(predicted_trajectory):
    traj = predicted_trajectory[..., :2]
    B, C, T = traj.shape[:3]
    x = traj[..., 0]
    y = traj[..., 1]
    W = _R * T
    nc = C // _R
    xf = x.reshape(B, nc, W)
    yf = y.reshape(B, nc, W)
    total = pl.pallas_call(
        _diversity_sum_kernel,
        grid=(B // _G,),
        in_specs=[
            pl.BlockSpec((_G, C, T), lambda b: (b, 0, 0)),
            pl.BlockSpec((_G, C, T), lambda b: (b, 0, 0)),
            pl.BlockSpec((_G, nc, W), lambda b: (b, 0, 0)),
            pl.BlockSpec((_G, nc, W), lambda b: (b, 0, 0)),
        ],
        out_specs=pl.BlockSpec((1, 1), lambda b: (0, 0)),
        out_shape=jax.ShapeDtypeStruct((1, 1), jnp.float32),
    )(x, y, xf, yf)[0, 0]
    offdiag = total / T - B * C * jnp.sqrt(jnp.float32(_EPS))
    return -(offdiag / (B * C * (C - 1)))


# speedup vs baseline: 3.6094x; 1.0029x over previous
import jax
import jax.numpy as jnp
from jax.experimental import pallas as pl

_EPS = 1e-9
_R = 8
_G = 16


def _diversity_sum_kernel(x_ref, y_ref, xf_ref, yf_ref, out_ref):
    b = pl.program_id(0)
    x = x_ref[...]
    y = y_ref[...]
    xf = xf_ref[...]
    yf = yf_ref[...]
    xt = jnp.concatenate([x] * _R, axis=2)
    yt = jnp.concatenate([y] * _R, axis=2)
    G, C, W = xt.shape
    nc = C // _R
    acc1 = jnp.zeros((G, _R, W), jnp.float32)
    acc2 = jnp.zeros((G, _R, W), jnp.float32)
    for cj in range(nc):
        xj = xf[:, cj:cj + 1, :]
        yj = yf[:, cj:cj + 1, :]
        for ci in range(cj + 1):
            dx = xt[:, ci * _R:(ci + 1) * _R, :] - xj
            dy = yt[:, ci * _R:(ci + 1) * _R, :] - yj
            s2 = dx * dx + dy * dy + _EPS
            d = s2 * jax.lax.rsqrt(s2)
            if ci == cj:
                acc1 = acc1 + d
            else:
                acc2 = acc2 + d
    s = 2.0 * jnp.sum(acc2) + jnp.sum(acc1)

    @pl.when(b == 0)
    def _():
        out_ref[:, :] = jnp.zeros_like(out_ref)

    out_ref[:, :] = out_ref[:, :] + s


def kernel(predicted_trajectory):
    traj = predicted_trajectory[..., :2]
    B, C, T = traj.shape[:3]
    x = traj[..., 0]
    y = traj[..., 1]
    W = _R * T
    nc = C // _R
    xf = x.reshape(B, nc, W)
    yf = y.reshape(B, nc, W)
    total = pl.pallas_call(
        _diversity_sum_kernel,
        grid=(B // _G,),
        in_specs=[
            pl.BlockSpec((_G, C, T), lambda b: (b, 0, 0)),
            pl.BlockSpec((_G, C, T), lambda b: (b, 0, 0)),
            pl.BlockSpec((_G, nc, W), lambda b: (b, 0, 0)),
            pl.BlockSpec((_G, nc, W), lambda b: (b, 0, 0)),
        ],
        out_specs=pl.BlockSpec((1, 1), lambda b: (0, 0)),
        out_shape=jax.ShapeDtypeStruct((1, 1), jnp.float32),
    )(x, y, xf, yf)[0, 0]
    offdiag = total / T - B * C * jnp.sqrt(jnp.float32(_EPS))
    return -(offdiag / (B * C * (C - 1)))


# speedup vs baseline: 5.1544x; 1.4280x over previous
import jax
import jax.numpy as jnp
from jax.experimental import pallas as pl
from jax.experimental.pallas import tpu as pltpu

_EPS = 1e-9
_R = 8
_G = 16


def _diversity_kernel(tr_ref, out_ref, *, T, scale):
    b = pl.program_id(0)
    nsteps = pl.num_programs(0)
    v = tr_ref[...]
    xf = v[0]
    yf = v[1]
    G, nc, W = xf.shape
    acc1 = jnp.zeros((G, nc, W), jnp.float32)
    acc2 = jnp.zeros((G, nc, W), jnp.float32)
    half = _R // 2
    for lc in range(half + 1):
        if lc == 0:
            xl, yl = xf, yf
            rcs = [(1, 2), (2, 2), (3, 2), (4, 1)]
        else:
            xl = pltpu.roll(xf, (_R - lc) * T, axis=2)
            yl = pltpu.roll(yf, (_R - lc) * T, axis=2)
            if lc == half:
                rcs = [(0, 1), (1, 2), (2, 2), (3, 2), (4, 1)]
            else:
                rcs = [(rc, 2) for rc in range(nc)]
        for rc, w in rcs:
            if rc == 0:
                xr, yr = xl, yl
            else:
                xr = pltpu.roll(xl, nc - rc, axis=1)
                yr = pltpu.roll(yl, nc - rc, axis=1)
            dx = xf - xr
            dy = yf - yr
            s2 = dx * dx + dy * dy + _EPS
            d = s2 * jax.lax.rsqrt(s2)
            if w == 1:
                acc1 = acc1 + d
            else:
                acc2 = acc2 + d
    s = 2.0 * jnp.sum(acc2) + jnp.sum(acc1)

    @pl.when(b == 0)
    def _():
        out_ref[:, :] = jnp.zeros_like(out_ref)

    out_ref[:, :] = out_ref[:, :] + s

    @pl.when(b == nsteps - 1)
    def _():
        out_ref[:, :] = out_ref[:, :] * (-scale)


def kernel(predicted_trajectory):
    B, C, T, _ = predicted_trajectory.shape
    nc = C // _R
    W = _R * T
    tp = jnp.moveaxis(predicted_trajectory, 3, 0).reshape(2, B, nc, W)
    import functools
    scale = 1.0 / (T * B * C * (C - 1))
    out = pl.pallas_call(
        functools.partial(_diversity_kernel, T=T, scale=scale),
        grid=(B // _G,),
        in_specs=[pl.BlockSpec((2, _G, nc, W), lambda b: (0, b, 0, 0))],
        out_specs=pl.BlockSpec((1, 1), lambda b: (0, 0)),
        out_shape=jax.ShapeDtypeStruct((1, 1), jnp.float32),
    )(tp)
    return out[0, 0]


# speedup vs baseline: 5.1856x; 1.0060x over previous
import jax
import jax.numpy as jnp
from jax.experimental import pallas as pl
from jax.experimental.pallas import tpu as pltpu

_EPS = 1e-9
_R = 8
_G = 32


def _diversity_kernel(tr_ref, out_ref, *, T, scale):
    b = pl.program_id(0)
    nsteps = pl.num_programs(0)
    v = tr_ref[...]
    xf = v[0]
    yf = v[1]
    G, nc, W = xf.shape
    acc1 = jnp.zeros((G, nc, W), jnp.float32)
    acc2 = jnp.zeros((G, nc, W), jnp.float32)
    half = _R // 2
    for lc in range(half + 1):
        if lc == 0:
            xl, yl = xf, yf
            rcs = [(1, 2), (2, 2), (3, 2), (4, 1)]
        else:
            xl = pltpu.roll(xf, (_R - lc) * T, axis=2)
            yl = pltpu.roll(yf, (_R - lc) * T, axis=2)
            if lc == half:
                rcs = [(0, 1), (1, 2), (2, 2), (3, 2), (4, 1)]
            else:
                rcs = [(rc, 2) for rc in range(nc)]
        for rc, w in rcs:
            if rc == 0:
                xr, yr = xl, yl
            else:
                xr = pltpu.roll(xl, nc - rc, axis=1)
                yr = pltpu.roll(yl, nc - rc, axis=1)
            dx = xf - xr
            dy = yf - yr
            s2 = dx * dx + dy * dy + _EPS
            d = s2 * jax.lax.rsqrt(s2)
            if w == 1:
                acc1 = acc1 + d
            else:
                acc2 = acc2 + d
    s = 2.0 * jnp.sum(acc2) + jnp.sum(acc1)

    @pl.when(b == 0)
    def _():
        out_ref[:, :] = jnp.zeros_like(out_ref)

    out_ref[:, :] = out_ref[:, :] + s

    @pl.when(b == nsteps - 1)
    def _():
        out_ref[:, :] = out_ref[:, :] * (-scale)


def kernel(predicted_trajectory):
    B, C, T, _ = predicted_trajectory.shape
    nc = C // _R
    W = _R * T
    tp = jnp.moveaxis(predicted_trajectory, 3, 0).reshape(2, B, nc, W)
    import functools
    scale = 1.0 / (T * B * C * (C - 1))
    out = pl.pallas_call(
        functools.partial(_diversity_kernel, T=T, scale=scale),
        grid=(B // _G,),
        in_specs=[pl.BlockSpec((2, _G, nc, W), lambda b: (0, b, 0, 0))],
        out_specs=pl.BlockSpec((1, 1), lambda b: (0, 0)),
        out_shape=jax.ShapeDtypeStruct((1, 1), jnp.float32),
    )(tp)
    return out[0, 0]
